# baseline (speedup 1.0000x reference)
import functools

import jax
import jax.numpy as jnp
from jax import lax
from jax.experimental import pallas as pl
from jax.experimental.pallas import tpu as pltpu
from jax.experimental.pallas import tpu_sc as plsc

D = 128
V0, V1, V2 = 5, 6, 2
VT = V0 * V1 * V2

NC, NS = 2, 16
NW = NC * NS

C = 64
NSLOT = 2


def _sc_body(bpw, w0_hbm, w1_hbm, w2_hbm, ea_hbm, out_hbm, w0_v, w1_v, w2_v,
             t_v, ea_v, rows_v, wsem0, wsem1):
    wid = lax.axis_index("s") * NC + lax.axis_index("c")
    base = wid * bpw

    pltpu.sync_copy(w0_hbm, w0_v)
    pltpu.sync_copy(w1_hbm, w1_v)
    pltpu.sync_copy(w2_hbm, w2_v)
    pltpu.sync_copy(ea_hbm.at[pl.ds(base * 3, bpw * 3)], ea_v)
    for r in range(VT):
        i0, i1, i2 = r // (V1 * V2), (r // V2) % V1, r % V2
        for c in range(D // 16):
            t_v[pl.ds(r * D + c * 16, 16)] = (
                w0_v[i0, pl.ds(c * 16, 16)]
                + w1_v[i1, pl.ds(c * 16, 16)]
                + w2_v[i2, pl.ds(c * 16, 16)]
            )

    iota = lax.iota(jnp.int32, 16)
    _dnums = lax.GatherDimensionNumbers(
        offset_dims=(), collapsed_slice_dims=(0,), start_index_map=(0,))

    def perm(vec, lane_vec):
        return lax.gather(
            vec, lane_vec[:, None], _dnums, slice_sizes=(1,),
            mode=lax.GatherScatterMode.PROMISE_IN_BOUNDS)

    part_lane, part_reg = [], []
    for t in range(3):
        f = iota * 3 + t
        part_lane.append(lax.rem(f, jnp.int32(16)))
        part_reg.append(f // 16)

    def weights_for(k):
        m = lax.rem(iota + (16 * k) % 3, jnp.int32(3))
        return jnp.where(m == 0, V1 * V2, jnp.where(m == 1, V2, 1))

    wts = [weights_for(k) for k in range(3)]

    def group_vec(eo):
        f0 = eo * 3
        m = [ea_v[pl.ds(f0 + 16 * k, 16)] * wts[k] for k in range(3)]
        civ = None
        for t in range(3):
            perms = [perm(m[k], part_lane[t]) for k in range(3)]
            part = jnp.where(part_reg[t] == 0, perms[0],
                             jnp.where(part_reg[t] == 1, perms[1], perms[2]))
            civ = part if civ is None else civ + part
        return civ * D

    def group_offsets(eo):
        civ = group_vec(eo)
        return [civ[l] for l in range(16)]

    n_blk = bpw // C
    tail = bpw - n_blk * C
    assert n_blk % NSLOT == 0 and tail % 16 == 0
    CB = NSLOT * C

    def wait_write(sem, slot, o):
        pltpu.make_async_copy(
            rows_v.at[pl.ds(slot * C, C)],
            out_hbm.at[pl.ds(base + lax.max(o, 0) * C, C)],
            sem,
        ).wait()

    def fire_write(sem, slot, o):
        pltpu.async_copy(
            rows_v.at[pl.ds(slot * C, C)],
            out_hbm.at[pl.ds(base + o * C, C)],
            sem,
        )

    def pair_body(p, carry):
        @pl.when(p > 0)
        def _():
            wait_write(wsem0, 0, (p - 1) * NSLOT)

        prev = None
        cis = group_offsets(p * CB)
        civ_next = None
        cis_next = None
        n_grp_pair = CB // 16
        for e in range(CB):
            g, ph = e // 16, e % 16
            if ph == 0 and g > 0:
                cis = cis_next
            if ph == 2 and g + 1 < n_grp_pair:
                civ_next = group_vec(p * CB + (g + 1) * 16)
            if ph == 8 and g + 1 < n_grp_pair:
                cis_next = [civ_next[l] for l in range(16)]
            if e == C:
                @pl.when(p > 0)
                def _():
                    wait_write(wsem1, 1, (p - 1) * NSLOT + 1)
            loads = []
            for c in range(D // 16):
                loads.append(t_v[pl.ds(cis[ph] + c * 16, 16)])
                if prev is not None:
                    pv_e, pv = prev
                    rows_v[pv_e, pl.ds(c * 16, 16)] = pv[c]
            prev = (e, loads)
            if e == C:
                fire_write(wsem0, 0, p * NSLOT)
        pv_e, pv = prev
        for c in range(D // 16):
            rows_v[pv_e, pl.ds(c * 16, 16)] = pv[c]
        fire_write(wsem1, 1, p * NSLOT + 1)
        return carry

    lax.fori_loop(0, n_blk // NSLOT, pair_body, 0)
    wait_write(wsem0, 0, n_blk - NSLOT)
    wait_write(wsem1, 1, n_blk - NSLOT + 1)

    if tail:
        for g in range(tail // 16):
            cis = group_offsets(n_blk * C + g * 16)
            for l in range(16):
                for c in range(D // 16):
                    rows_v[g * 16 + l, pl.ds(c * 16, 16)] = (
                        t_v[pl.ds(cis[l] + c * 16, 16)])
        pltpu.sync_copy(
            rows_v.at[pl.ds(0, tail)],
            out_hbm.at[pl.ds(base + n_blk * C, tail)],
        )


def kernel(edge_attr, W0, W1, W2):
    E = edge_attr.shape[0]
    bpw = E // NW
    assert E == bpw * NW and bpw % 16 == 0

    ea_flat = edge_attr.astype(jnp.int32).reshape(-1)

    mesh = plsc.VectorSubcoreMesh(core_axis_name="c", subcore_axis_name="s")
    sc_kernel = functools.partial(
        pl.kernel,
        out_type=jax.ShapeDtypeStruct((E, D), jnp.float32),
        mesh=mesh,
        compiler_params=pltpu.CompilerParams(needs_layout_passes=False),
        scratch_types=[
            pltpu.VMEM((V0, D), jnp.float32),
            pltpu.VMEM((V1, D), jnp.float32),
            pltpu.VMEM((V2, D), jnp.float32),
            pltpu.VMEM((VT * D,), jnp.float32),
            pltpu.VMEM((bpw * 3,), jnp.int32),
            pltpu.VMEM((NSLOT * C, D), jnp.float32),
            pltpu.SemaphoreType.DMA,
            pltpu.SemaphoreType.DMA,
        ],
    )(functools.partial(_sc_body, bpw))
    return sc_kernel(W0, W1, W2, ea_flat)


# speedup vs baseline: 1.0174x; 1.0174x over previous
import functools

import jax
import jax.numpy as jnp
from jax import lax
from jax.experimental import pallas as pl
from jax.experimental.pallas import tpu as pltpu
from jax.experimental.pallas import tpu_sc as plsc

D = 128
V0, V1, V2 = 5, 6, 2
VT = V0 * V1 * V2

NC, NS = 2, 16
NW = NC * NS

C = 64
NSLOT = 2


def _sc_body(bpw, w0_hbm, w1_hbm, w2_hbm, ea_hbm, out_hbm, w0_v, w1_v, w2_v,
             t_v, ea_v, rows_v, wsem0, wsem1):
    wid = lax.axis_index("s") * NC + lax.axis_index("c")
    base = wid * bpw

    pltpu.sync_copy(w0_hbm, w0_v)
    pltpu.sync_copy(w1_hbm, w1_v)
    pltpu.sync_copy(w2_hbm, w2_v)
    pltpu.sync_copy(ea_hbm.at[pl.ds(base * 3, bpw * 3)], ea_v)

    def table_body(r, carry):
        i0 = (r // (V1 * V2)) * D
        i1 = ((r // V2) % V1) * D
        i2 = (r % V2) * D
        for c in range(D // 16):
            t_v[pl.ds(r * D + c * 16, 16)] = (
                w0_v[pl.ds(i0 + c * 16, 16)]
                + w1_v[pl.ds(i1 + c * 16, 16)]
                + w2_v[pl.ds(i2 + c * 16, 16)]
            )
        return carry

    lax.fori_loop(0, VT, table_body, 0)

    iota = lax.iota(jnp.int32, 16)
    _dnums = lax.GatherDimensionNumbers(
        offset_dims=(), collapsed_slice_dims=(0,), start_index_map=(0,))

    def perm(vec, lane_vec):
        return lax.gather(
            vec, lane_vec[:, None], _dnums, slice_sizes=(1,),
            mode=lax.GatherScatterMode.PROMISE_IN_BOUNDS)

    part_lane, part_reg = [], []
    for t in range(3):
        f = iota * 3 + t
        part_lane.append(lax.rem(f, jnp.int32(16)))
        part_reg.append(f // 16)

    def weights_for(k):
        m = lax.rem(iota + (16 * k) % 3, jnp.int32(3))
        return jnp.where(m == 0, V1 * V2, jnp.where(m == 1, V2, 1))

    wts = [weights_for(k) for k in range(3)]

    def group_vec(eo):
        f0 = eo * 3
        m = [ea_v[pl.ds(f0 + 16 * k, 16)] * wts[k] for k in range(3)]
        civ = None
        for t in range(3):
            perms = [perm(m[k], part_lane[t]) for k in range(3)]
            part = jnp.where(part_reg[t] == 0, perms[0],
                             jnp.where(part_reg[t] == 1, perms[1], perms[2]))
            civ = part if civ is None else civ + part
        return civ * D

    def group_offsets(eo):
        civ = group_vec(eo)
        return [civ[l] for l in range(16)]

    n_blk = bpw // C
    tail = bpw - n_blk * C
    assert n_blk % NSLOT == 0 and tail % 16 == 0
    CB = NSLOT * C

    def wait_write(sem, slot, o):
        pltpu.make_async_copy(
            rows_v.at[pl.ds(slot * C, C)],
            out_hbm.at[pl.ds(base + lax.max(o, 0) * C, C)],
            sem,
        ).wait()

    def fire_write(sem, slot, o):
        pltpu.async_copy(
            rows_v.at[pl.ds(slot * C, C)],
            out_hbm.at[pl.ds(base + o * C, C)],
            sem,
        )

    def pair_body(p, carry):
        @pl.when(p > 0)
        def _():
            wait_write(wsem0, 0, (p - 1) * NSLOT)

        prev = None
        cis = group_offsets(p * CB)
        civ_next = None
        cis_next = None
        n_grp_pair = CB // 16
        for e in range(CB):
            g, ph = e // 16, e % 16
            if ph == 0 and g > 0:
                cis = cis_next
            if ph == 2 and g + 1 < n_grp_pair:
                civ_next = group_vec(p * CB + (g + 1) * 16)
            if ph == 8 and g + 1 < n_grp_pair:
                cis_next = [civ_next[l] for l in range(16)]
            if e == C:
                @pl.when(p > 0)
                def _():
                    wait_write(wsem1, 1, (p - 1) * NSLOT + 1)
            loads = []
            for c in range(D // 16):
                loads.append(t_v[pl.ds(cis[ph] + c * 16, 16)])
                if prev is not None:
                    pv_e, pv = prev
                    rows_v[pv_e, pl.ds(c * 16, 16)] = pv[c]
            prev = (e, loads)
            if e == C:
                fire_write(wsem0, 0, p * NSLOT)
        pv_e, pv = prev
        for c in range(D // 16):
            rows_v[pv_e, pl.ds(c * 16, 16)] = pv[c]
        fire_write(wsem1, 1, p * NSLOT + 1)
        return carry

    lax.fori_loop(0, n_blk // NSLOT, pair_body, 0)
    wait_write(wsem0, 0, n_blk - NSLOT)
    wait_write(wsem1, 1, n_blk - NSLOT + 1)

    if tail:
        for g in range(tail // 16):
            cis = group_offsets(n_blk * C + g * 16)
            for l in range(16):
                for c in range(D // 16):
                    rows_v[g * 16 + l, pl.ds(c * 16, 16)] = (
                        t_v[pl.ds(cis[l] + c * 16, 16)])
        pltpu.sync_copy(
            rows_v.at[pl.ds(0, tail)],
            out_hbm.at[pl.ds(base + n_blk * C, tail)],
        )


def kernel(edge_attr, W0, W1, W2):
    E = edge_attr.shape[0]
    bpw = E // NW
    assert E == bpw * NW and bpw % 16 == 0

    ea_flat = edge_attr.astype(jnp.int32).reshape(-1)

    mesh = plsc.VectorSubcoreMesh(core_axis_name="c", subcore_axis_name="s")
    sc_kernel = functools.partial(
        pl.kernel,
        out_type=jax.ShapeDtypeStruct((E, D), jnp.float32),
        mesh=mesh,
        compiler_params=pltpu.CompilerParams(needs_layout_passes=False),
        scratch_types=[
            pltpu.VMEM((V0 * D,), jnp.float32),
            pltpu.VMEM((V1 * D,), jnp.float32),
            pltpu.VMEM((V2 * D,), jnp.float32),
            pltpu.VMEM((VT * D,), jnp.float32),
            pltpu.VMEM((bpw * 3,), jnp.int32),
            pltpu.VMEM((NSLOT * C, D), jnp.float32),
            pltpu.SemaphoreType.DMA,
            pltpu.SemaphoreType.DMA,
        ],
    )(functools.partial(_sc_body, bpw))
    return sc_kernel(W0.reshape(-1), W1.reshape(-1), W2.reshape(-1), ea_flat)


# speedup vs baseline: 2.8081x; 2.7599x over previous
import functools

import jax
import jax.numpy as jnp
from jax import lax
from jax.experimental import pallas as pl
from jax.experimental.pallas import tpu as pltpu
from jax.experimental.pallas import tpu_sc as plsc

D = 128
V0, V1, V2 = 5, 6, 2
VT = V0 * V1 * V2

NC, NS = 2, 16
NW = NC * NS

C = 64
NSLOT = 2


def _sc_body(bpw, w0_hbm, w1_hbm, w2_hbm, ea0_hbm, ea1_hbm, ea2_hbm, out_hbm,
             w0_v, w1_v, w2_v, t_v, ea0_v, ea1_v, ea2_v, rows_v, wsem0, wsem1):
    wid = lax.axis_index("s") * NC + lax.axis_index("c")
    base = wid * bpw

    pltpu.sync_copy(w0_hbm, w0_v)
    pltpu.sync_copy(w1_hbm, w1_v)
    pltpu.sync_copy(w2_hbm, w2_v)
    pltpu.sync_copy(ea0_hbm.at[pl.ds(base, bpw)], ea0_v)
    pltpu.sync_copy(ea1_hbm.at[pl.ds(base, bpw)], ea1_v)
    pltpu.sync_copy(ea2_hbm.at[pl.ds(base, bpw)], ea2_v)

    def table_body(r, carry):
        i0 = (r // (V1 * V2)) * D
        i1 = ((r // V2) % V1) * D
        i2 = (r % V2) * D
        for c in range(D // 16):
            t_v[pl.ds(r * D + c * 16, 16)] = (
                w0_v[pl.ds(i0 + c * 16, 16)]
                + w1_v[pl.ds(i1 + c * 16, 16)]
                + w2_v[pl.ds(i2 + c * 16, 16)]
            )
        return carry

    lax.fori_loop(0, VT, table_body, 0)

    iota = lax.iota(jnp.int32, 16)

    def group_vec(eo):
        i0 = ea0_v[pl.ds(eo, 16)]
        i1 = ea1_v[pl.ds(eo, 16)]
        i2 = ea2_v[pl.ds(eo, 16)]
        return (i0 * (V1 * V2) + i1 * V2 + i2) * D

    def group_offsets(eo):
        civ = group_vec(eo)
        return [civ[l] for l in range(16)]

    n_blk = bpw // C
    tail = bpw - n_blk * C
    assert n_blk % NSLOT == 0 and tail % 16 == 0
    CB = NSLOT * C

    def wait_write(sem, slot, o):
        pltpu.make_async_copy(
            rows_v.at[pl.ds(slot * C, C)],
            out_hbm.at[pl.ds(base + lax.max(o, 0) * C, C)],
            sem,
        ).wait()

    def fire_write(sem, slot, o):
        pltpu.async_copy(
            rows_v.at[pl.ds(slot * C, C)],
            out_hbm.at[pl.ds(base + o * C, C)],
            sem,
        )

    def pair_body(p, carry):
        @pl.when(p > 0)
        def _():
            wait_write(wsem0, 0, (p - 1) * NSLOT)

        prev = None
        cis = group_offsets(p * CB)
        civ_next = None
        cis_next = None
        n_grp_pair = CB // 16
        for e in range(CB):
            g, ph = e // 16, e % 16
            if ph == 0 and g > 0:
                cis = cis_next
            if ph == 2 and g + 1 < n_grp_pair:
                civ_next = group_vec(p * CB + (g + 1) * 16)
            if ph == 8 and g + 1 < n_grp_pair:
                cis_next = [civ_next[l] for l in range(16)]
            if e == C:
                @pl.when(p > 0)
                def _():
                    wait_write(wsem1, 1, (p - 1) * NSLOT + 1)
            loads = []
            for c in range(D // 16):
                loads.append(t_v[pl.ds(cis[ph] + c * 16, 16)])
                if prev is not None:
                    pv_e, pv = prev
                    rows_v[pv_e, pl.ds(c * 16, 16)] = pv[c]
            prev = (e, loads)
            if e == C:
                fire_write(wsem0, 0, p * NSLOT)
        pv_e, pv = prev
        for c in range(D // 16):
            rows_v[pv_e, pl.ds(c * 16, 16)] = pv[c]
        fire_write(wsem1, 1, p * NSLOT + 1)
        return carry

    lax.fori_loop(0, n_blk // NSLOT, pair_body, 0)
    wait_write(wsem0, 0, n_blk - NSLOT)
    wait_write(wsem1, 1, n_blk - NSLOT + 1)

    if tail:
        for g in range(tail // 16):
            cis = group_offsets(n_blk * C + g * 16)
            for l in range(16):
                for c in range(D // 16):
                    rows_v[g * 16 + l, pl.ds(c * 16, 16)] = (
                        t_v[pl.ds(cis[l] + c * 16, 16)])
        pltpu.sync_copy(
            rows_v.at[pl.ds(0, tail)],
            out_hbm.at[pl.ds(base + n_blk * C, tail)],
        )


def kernel(edge_attr, W0, W1, W2):
    E = edge_attr.shape[0]
    bpw = E // NW
    assert E == bpw * NW and bpw % 16 == 0

    ea = edge_attr.astype(jnp.int32)
    ea0, ea1, ea2 = ea[:, 0], ea[:, 1], ea[:, 2]

    mesh = plsc.VectorSubcoreMesh(core_axis_name="c", subcore_axis_name="s")
    sc_kernel = functools.partial(
        pl.kernel,
        out_type=jax.ShapeDtypeStruct((E, D), jnp.float32),
        mesh=mesh,
        compiler_params=pltpu.CompilerParams(needs_layout_passes=False),
        scratch_types=[
            pltpu.VMEM((V0 * D,), jnp.float32),
            pltpu.VMEM((V1 * D,), jnp.float32),
            pltpu.VMEM((V2 * D,), jnp.float32),
            pltpu.VMEM((VT * D,), jnp.float32),
            pltpu.VMEM((bpw,), jnp.int32),
            pltpu.VMEM((bpw,), jnp.int32),
            pltpu.VMEM((bpw,), jnp.int32),
            pltpu.VMEM((NSLOT * C, D), jnp.float32),
            pltpu.SemaphoreType.DMA,
            pltpu.SemaphoreType.DMA,
        ],
    )(functools.partial(_sc_body, bpw))
    return sc_kernel(W0.reshape(-1), W1.reshape(-1), W2.reshape(-1),
                     ea0, ea1, ea2)
